# baseline (speedup 1.0000x reference)
import functools

import jax
import jax.numpy as jnp
from jax import lax
from jax.experimental import pallas as pl
from jax.experimental.pallas import tpu as pltpu
from jax.experimental.pallas import tpu_sc as plsc

_B, _H, _L, _D, _S = 8, 16, 2048, 128, 16
_BH = _B * _H
_GB = 4

_NC, _NS = 2, 16
_NW = _NC * _NS
_BH_PER_W = _BH // _NW
_CHUNK = 256
_NCH = _L // _CHUNK
_NBUF = 3


def _tc_body(pos_ref, kc_ref, kv_ref, ko_ref):
    ko_ref[...] = kc_ref[...]
    for i in range(_S):
        p = pos_ref[i]
        for j in range(_GB):
            ko_ref[j, pl.ds(p, 1), :] = kv_ref[j, pl.ds(i, 1), :]


def _tc_update(input_pos, val, cache):
    cache_spec = pl.BlockSpec((_GB, _L, _D), lambda i: (i, 0, 0))
    val_spec = pl.BlockSpec((_GB, _S, _D), lambda i: (i, 0, 0))
    out = pl.pallas_call(
        _tc_body,
        grid=(_BH // _GB,),
        in_specs=[
            pl.BlockSpec(memory_space=pltpu.SMEM),
            cache_spec,
            val_spec,
        ],
        out_specs=cache_spec,
        out_shape=jax.ShapeDtypeStruct((_BH, _L, _D), jnp.float32),
        compiler_params=pltpu.CompilerParams(
            dimension_semantics=("arbitrary",),
        ),
    )(input_pos, cache, val)
    return out


def _sc_tec_body(pos_hbm, val_hbm, cache_hbm, out_hbm,
                 pos_v, idx_v, rows_v, *rest):
    bufs = rest[:_NBUF]
    sem_r = rest[_NBUF:2 * _NBUF]
    sem_w = rest[2 * _NBUF:3 * _NBUF]
    sem_s = rest[3 * _NBUF]
    wid = lax.axis_index("s") * _NC + lax.axis_index("c")
    base_bh = wid * _BH_PER_W

    pltpu.sync_copy(pos_hbm, pos_v)

    chunks = [(j, c) for j in range(_BH_PER_W) for c in range(_NCH)]
    T = len(chunks)

    def row_slice(t):
        j, c = chunks[t]
        start = (base_bh + j) * _L + c * _CHUNK
        return pl.ds(start, _CHUNK)

    reads = [None] * T
    writes = [None] * T
    for t in range(min(_NBUF, T)):
        reads[t] = pltpu.async_copy(
            cache_hbm.at[row_slice(t)], bufs[t], sem_r[t])
    for t in range(T):
        slot = t % _NBUF
        reads[t].wait()
        writes[t] = pltpu.async_copy(
            bufs[slot], out_hbm.at[row_slice(t)], sem_w[slot])
        writes[t].wait()
        nxt = t + _NBUF
        if nxt < T:
            reads[nxt] = pltpu.async_copy(
                cache_hbm.at[row_slice(nxt)], bufs[slot], sem_r[slot])

    pv = pos_v[...]
    for j in range(_BH_PER_W):
        bh = base_bh + j
        idx_v[...] = pv + bh * _L
        pltpu.sync_copy(val_hbm.at[pl.ds(bh * _S, _S)], rows_v)
        pltpu.async_copy(rows_v, out_hbm.at[idx_v], sem_s).wait()


def _sc_update(input_pos, val, cache):
    mesh = plsc.VectorSubcoreMesh(
        core_axis_name="c", subcore_axis_name="s",
        num_cores=_NC, num_subcores=_NS)
    scratch = (
        [pltpu.VMEM((_S,), jnp.int32),
         pltpu.VMEM((_S,), jnp.int32),
         pltpu.VMEM((_S, _D), jnp.float32)]
        + [pltpu.VMEM((_CHUNK, _D), jnp.float32) for _ in range(_NBUF)]
        + [pltpu.SemaphoreType.DMA for _ in range(2 * _NBUF + 1)]
    )
    run = pl.kernel(
        _sc_tec_body,
        out_type=jax.ShapeDtypeStruct((_BH * _L, _D), jnp.float32),
        mesh=mesh,
        scratch_types=scratch,
    )
    return run(input_pos, val.reshape(_BH * _S, _D),
               cache.reshape(_BH * _L, _D))


def kernel(input_pos, k_val, v_val, k_cache, v_cache):
    kc = k_cache.reshape(_BH, _L, _D)
    kv = k_val.reshape(_BH, _S, _D)
    ko = _tc_update(input_pos, kv, kc)
    vo = _sc_update(input_pos, v_val, v_cache)
    return (ko.reshape(_B, _H, _L, _D), vo.reshape(_B, _H, _L, _D))
